# baseline (speedup 1.0000x reference)
import functools

import jax
import jax.numpy as jnp
from jax import lax
from jax.experimental import pallas as pl
from jax.experimental.pallas import tpu as pltpu, tpu_sc as plsc

M, P, D = 16, 1024, 768
NC, NS, L = 2, 16, 16
NW = NC * NS
PPW = P // NW
SL = D // L

_mesh = plsc.VectorSubcoreMesh(core_axis_name="c", subcore_axis_name="s")


@functools.partial(
    pl.kernel,
    out_type=jax.ShapeDtypeStruct((M, P, D), jnp.float32),
    mesh=_mesh,
    scratch_types=[
        pltpu.VMEM((PPW, D), jnp.float32),
        pltpu.VMEM((PPW, D), jnp.float32),
        pltpu.VMEM((PPW, M), jnp.float32),
        pltpu.VMEM((PPW, D), jnp.float32),
    ],
)
def _sc_kernel(feat_hbm, pos_hbm, maskT_hbm, out_hbm, a_v, b_v, mask_v, ob_v):
    wid = lax.axis_index("s") * NC + lax.axis_index("c")
    base = wid * PPW
    pltpu.sync_copy(feat_hbm.at[pl.ds(base, PPW)], a_v)
    pltpu.sync_copy(pos_hbm.at[pl.ds(base, PPW)], b_v)
    pltpu.sync_copy(maskT_hbm.at[pl.ds(base, PPW)], mask_v)

    def add_row(p, carry):
        for j in range(SL):
            sl = pl.ds(j * L, L)
            a_v[p, sl] = a_v[p, sl] + b_v[p, sl]
        return carry

    lax.fori_loop(0, PPW, add_row, 0)

    for m in range(M):
        def row(p, c, m=m):
            mval = mask_v[p, :][m]
            for j in range(SL):
                sl = pl.ds(j * L, L)
                ob_v[p, sl] = a_v[p, sl] * mval
            return c

        lax.fori_loop(0, PPW, row, 0)
        pltpu.sync_copy(ob_v, out_hbm.at[m, pl.ds(base, PPW)])


def kernel(image_features, pos_table, masks):
    maskT = masks.T.astype(jnp.float32)
    return _sc_kernel(image_features, pos_table, maskT)


# speedup vs baseline: 1.2761x; 1.2761x over previous
import functools

import jax
import jax.numpy as jnp
from jax import lax
from jax.experimental import pallas as pl
from jax.experimental.pallas import tpu as pltpu, tpu_sc as plsc

M, P, D = 16, 1024, 768
NC, NS, L = 2, 16, 16
NW = NC * NS
PPW = P // NW
SL = D // L

_mesh = plsc.VectorSubcoreMesh(core_axis_name="c", subcore_axis_name="s")


@functools.partial(
    pl.kernel,
    out_type=jax.ShapeDtypeStruct((M, P, D), jnp.float32),
    mesh=_mesh,
    scratch_types=[
        pltpu.VMEM((PPW, D), jnp.float32),
        pltpu.VMEM((PPW, D), jnp.float32),
        pltpu.VMEM((PPW, M), jnp.float32),
        pltpu.VMEM((PPW, D), jnp.float32),
        pltpu.VMEM((PPW, D), jnp.float32),
        pltpu.SemaphoreType.DMA,
        pltpu.SemaphoreType.DMA,
    ],
)
def _sc_kernel(feat_hbm, pos_hbm, maskT_hbm, out_hbm,
               a_v, b_v, mask_v, ob0_v, ob1_v, sem0, sem1):
    wid = lax.axis_index("s") * NC + lax.axis_index("c")
    base = wid * PPW
    pltpu.sync_copy(feat_hbm.at[pl.ds(base, PPW)], a_v)
    pltpu.sync_copy(pos_hbm.at[pl.ds(base, PPW)], b_v)
    pltpu.sync_copy(maskT_hbm.at[pl.ds(base, PPW)], mask_v)

    def add_row(p, carry):
        for j in range(SL):
            sl = pl.ds(j * L, L)
            a_v[p, sl] = a_v[p, sl] + b_v[p, sl]
        return carry

    lax.fori_loop(0, PPW, add_row, 0)

    obufs = (ob0_v, ob1_v)
    sems = (sem0, sem1)
    pending = [None, None]
    for m in range(M):
        ob, sem = obufs[m % 2], sems[m % 2]
        if pending[m % 2] is not None:
            pending[m % 2].wait()

        def row(p, c, m=m, ob=ob):
            mval = mask_v[p, :][m]
            for j in range(SL):
                sl = pl.ds(j * L, L)
                ob[p, sl] = a_v[p, sl] * mval
            return c

        lax.fori_loop(0, PPW, row, 0)
        pending[m % 2] = pltpu.async_copy(ob, out_hbm.at[m, pl.ds(base, PPW)], sem)
    pending[0].wait()
    pending[1].wait()


def kernel(image_features, pos_table, masks):
    maskT = masks.T.astype(jnp.float32)
    return _sc_kernel(image_features, pos_table, maskT)
